# baseline (speedup 1.0000x reference)
import functools

import jax
import jax.numpy as jnp
import numpy as np
from jax import lax
from jax.experimental import pallas as pl
from jax.experimental.pallas import tpu as pltpu
from jax.experimental.pallas import tpu_sc as plsc

B = 32
P = 1024
_BN = np.float32(1.0 / np.sqrt(1.0 + 1e-5))
_R2_1 = np.float32(np.float64(0.2) * np.float64(0.2))
_R2_2 = np.float32(np.float64(0.4) * np.float64(0.4))


def _relu_bn(x):
    return jnp.maximum(x, 0.0) * _BN


_L = 16


def _fps_level(pn, npoint, xv, yv, zv, dv, cxv, cyv, czv):
    nchunk = pn // _L
    lane = lax.broadcasted_iota(jnp.int32, (_L,), 0)
    m0 = lane == 0
    zero = jnp.zeros((_L,), jnp.int32)
    for j in range(nchunk):
        dv[pl.ds(_L * j, _L)] = jnp.full((_L,), 1e10, jnp.float32)

    lx0 = plsc.load_gather(xv, [zero])
    ly0 = plsc.load_gather(yv, [zero])
    lz0 = plsc.load_gather(zv, [zero])
    plsc.store_scatter(cxv, [zero], lx0, mask=m0)
    plsc.store_scatter(cyv, [zero], ly0, mask=m0)
    plsc.store_scatter(czv, [zero], lz0, mask=m0)

    def outer(i, st):
        lxv, lyv, lzv = st
        vmax = jnp.full((_L,), -1.0, jnp.float32)
        vidx = jnp.zeros((_L,), jnp.int32)
        for j in range(nchunk):
            sl = pl.ds(_L * j, _L)
            dx = xv[sl] - lxv
            dy = yv[sl] - lyv
            dz = zv[sl] - lzv
            dd = (dx * dx + dy * dy) + dz * dz
            dn = jnp.minimum(dv[sl], dd)
            dv[sl] = dn
            better = dn > vmax
            vmax = jnp.where(better, dn, vmax)
            vidx = jnp.where(better, lane + _L * j, vidx)
        gm = jnp.max(vmax)
        biv = jnp.full((_L,), jnp.min(jnp.where(vmax == gm, vidx, pn)))
        nlx = plsc.load_gather(xv, [biv])
        nly = plsc.load_gather(yv, [biv])
        nlz = plsc.load_gather(zv, [biv])
        iv = jnp.full((_L,), i)
        plsc.store_scatter(cxv, [iv], nlx, mask=m0)
        plsc.store_scatter(cyv, [iv], nly, mask=m0)
        plsc.store_scatter(czv, [iv], nlz, mask=m0)
        return nlx, nly, nlz

    lax.fori_loop(1, npoint, outer, (lx0, ly0, lz0))


def _sc_fps_body(pn, n1, n2, x_hbm, y_hbm, z_hbm,
                 o1x_hbm, o1y_hbm, o1z_hbm, o2x_hbm, o2y_hbm, o2z_hbm,
                 xv, yv, zv, dv, c1x, c1y, c1z, d2v, c2x, c2y, c2z):
    wid = lax.axis_index("s") * 2 + lax.axis_index("c")
    pltpu.sync_copy(x_hbm.at[wid], xv)
    pltpu.sync_copy(y_hbm.at[wid], yv)
    pltpu.sync_copy(z_hbm.at[wid], zv)

    _fps_level(pn, n1, xv, yv, zv, dv, c1x, c1y, c1z)
    _fps_level(n1, n2, c1x, c1y, c1z, d2v, c2x, c2y, c2z)

    pltpu.sync_copy(c1x, o1x_hbm.at[wid])
    pltpu.sync_copy(c1y, o1y_hbm.at[wid])
    pltpu.sync_copy(c1z, o1z_hbm.at[wid])
    pltpu.sync_copy(c2x, o2x_hbm.at[wid])
    pltpu.sync_copy(c2y, o2y_hbm.at[wid])
    pltpu.sync_copy(c2z, o2z_hbm.at[wid])


def _fps2(x, y, z, n1, n2):
    pn = x.shape[1]
    out1 = jax.ShapeDtypeStruct((B, n1), jnp.float32)
    out2 = jax.ShapeDtypeStruct((B, n2), jnp.float32)
    mesh = plsc.VectorSubcoreMesh(core_axis_name="c", subcore_axis_name="s")
    fn = pl.kernel(
        functools.partial(_sc_fps_body, pn, n1, n2),
        out_type=(out1, out1, out1, out2, out2, out2),
        mesh=mesh,
        compiler_params=pltpu.CompilerParams(needs_layout_passes=False),
        scratch_types=[
            pltpu.VMEM((pn,), jnp.float32),
            pltpu.VMEM((pn,), jnp.float32),
            pltpu.VMEM((pn,), jnp.float32),
            pltpu.VMEM((pn,), jnp.float32),
            pltpu.VMEM((n1,), jnp.float32),
            pltpu.VMEM((n1,), jnp.float32),
            pltpu.VMEM((n1,), jnp.float32),
            pltpu.VMEM((n1,), jnp.float32),
            pltpu.VMEM((n2,), jnp.float32),
            pltpu.VMEM((n2,), jnp.float32),
            pltpu.VMEM((n2,), jnp.float32),
        ],
    )
    return fn(x, y, z)


def _sa_body(nc, pn, ns, r2, prow_ref, gfeat_ref, cmat_ref, lt_ref,
             wg_ref, wc_ref, b1_ref, w2_ref, b2_ref, w3_ref, b3_ref, out_ref):
    px = prow_ref[0, 0:1, :]
    py = prow_ref[0, 1:2, :]
    pz = prow_ref[0, 2:3, :]
    cm = cmat_ref[0]
    cx = cm[:, 0:1]
    cy = cm[:, 1:2]
    cz = cm[:, 2:3]
    d2 = ((cx - px) ** 2 + (cy - py) ** 2) + (cz - pz) ** 2
    mask = jnp.where(d2 <= r2, 1.0, 0.0)
    rank = jnp.dot(mask.astype(jnp.bfloat16),
                   lt_ref[...].astype(jnp.bfloat16),
                   preferred_element_type=jnp.float32)
    mrank = rank * mask
    count = rank[:, pn - 1 : pn]

    g_tab = jnp.dot(gfeat_ref[0], wg_ref[...],
                    preferred_element_type=jnp.float32)
    f1 = g_tab.shape[1]

    riota = lax.broadcasted_iota(jnp.int32, (1, ns, 1), 1).astype(
        jnp.float32) + 1.0
    sel = jnp.where(mrank[:, None, :] == riota, 1.0, 0.0)
    g_hi = g_tab.astype(jnp.bfloat16)
    g_lo = (g_tab - g_hi.astype(jnp.float32)).astype(jnp.bfloat16)
    sel2 = sel.reshape(nc * ns, pn).astype(jnp.bfloat16)
    gath = (jnp.dot(sel2, g_hi, preferred_element_type=jnp.float32)
            + jnp.dot(sel2, g_lo, preferred_element_type=jnp.float32))
    g3 = gath.reshape(nc, ns, f1)
    slot = lax.broadcasted_iota(jnp.int32, (nc, ns, 1), 1).astype(jnp.float32)
    g3 = jnp.where(slot < count[:, :, None], g3, g3[:, 0:1, :])

    cc = jnp.dot(cm, wc_ref[...], preferred_element_type=jnp.float32)
    h = _relu_bn(g3 - cc[:, None, :] + b1_ref[...][None])
    h = h.reshape(nc * ns, f1)
    h = _relu_bn(jnp.dot(h, w2_ref[...],
                         preferred_element_type=jnp.float32) + b2_ref[...])
    h = _relu_bn(jnp.dot(h, w3_ref[...],
                         preferred_element_type=jnp.float32) + b3_ref[...])
    f3 = h.shape[1]
    out_ref[0] = jnp.max(h.reshape(nc, ns, f3), axis=1)


def _sa_call(prow, gfeat, cmat, lt, wg, wc, b1, w2, b2, w3, b3,
             nc_block, ns, r2):
    b, pn, cg = gfeat.shape
    ncent = cmat.shape[1]
    nblk = ncent // nc_block
    f3 = w3.shape[1]
    grid = (b, nblk)
    return pl.pallas_call(
        functools.partial(_sa_body, nc_block, pn, ns, r2),
        grid=grid,
        in_specs=[
            pl.BlockSpec((1, 3, pn), lambda i, j: (i, 0, 0)),
            pl.BlockSpec((1, pn, cg), lambda i, j: (i, 0, 0)),
            pl.BlockSpec((1, nc_block, 8), lambda i, j: (i, j, 0)),
            pl.BlockSpec((pn, pn), lambda i, j: (0, 0)),
            pl.BlockSpec(wg.shape, lambda i, j: (0, 0)),
            pl.BlockSpec(wc.shape, lambda i, j: (0, 0)),
            pl.BlockSpec(b1.shape, lambda i, j: (0, 0)),
            pl.BlockSpec(w2.shape, lambda i, j: (0, 0)),
            pl.BlockSpec(b2.shape, lambda i, j: (0, 0)),
            pl.BlockSpec(w3.shape, lambda i, j: (0, 0)),
            pl.BlockSpec(b3.shape, lambda i, j: (0, 0)),
        ],
        out_specs=pl.BlockSpec((1, nc_block, f3), lambda i, j: (i, j, 0)),
        out_shape=jax.ShapeDtypeStruct((b, ncent, f3), jnp.float32),
    )(prow, gfeat, cmat, lt, wg, wc, b1, w2, b2, w3, b3)


def _sa3_body(wx_ref, wr_ref, b1_ref, w2_ref, b2_ref, w3_ref, b3_ref,
              x2_ref, cmat_ref, out_ref):
    h = jnp.dot(x2_ref[0], wx_ref[...], preferred_element_type=jnp.float32)
    h = h + jnp.dot(cmat_ref[0], wr_ref[...],
                    preferred_element_type=jnp.float32)
    h = _relu_bn(h + b1_ref[...])
    h = _relu_bn(jnp.dot(h, w2_ref[...],
                         preferred_element_type=jnp.float32) + b2_ref[...])
    h = _relu_bn(jnp.dot(h, w3_ref[...],
                         preferred_element_type=jnp.float32) + b3_ref[...])
    out_ref[0] = jnp.max(h, axis=0, keepdims=True)


def _sa3_call(x2, cmat, wx, wr, b1, w2, b2, w3, b3):
    b, n2, _ = x2.shape
    return pl.pallas_call(
        _sa3_body,
        grid=(b,),
        in_specs=[
            pl.BlockSpec(wx.shape, lambda i: (0, 0)),
            pl.BlockSpec(wr.shape, lambda i: (0, 0)),
            pl.BlockSpec(b1.shape, lambda i: (0, 0)),
            pl.BlockSpec(w2.shape, lambda i: (0, 0)),
            pl.BlockSpec(b2.shape, lambda i: (0, 0)),
            pl.BlockSpec(w3.shape, lambda i: (0, 0)),
            pl.BlockSpec(b3.shape, lambda i: (0, 0)),
            pl.BlockSpec((1, n2, x2.shape[2]), lambda i: (i, 0, 0)),
            pl.BlockSpec((1, n2, 8), lambda i: (i, 0, 0)),
        ],
        out_specs=pl.BlockSpec((1, 1, 1024), lambda i: (i, 0, 0)),
        out_shape=jax.ShapeDtypeStruct((b, 1, 1024), jnp.float32),
    )(wx, wr, b1, w2, b2, w3, b3, x2, cmat)


def _head_body(g_ref, w1_ref, b1_ref, w2_ref, b2_ref, w3_ref, b3_ref,
               out_ref):
    h = jnp.maximum((jnp.dot(g_ref[...], w1_ref[...],
                             preferred_element_type=jnp.float32)
                     + b1_ref[...]) * _BN, 0.0)
    h = jnp.maximum((jnp.dot(h, w2_ref[...],
                             preferred_element_type=jnp.float32)
                     + b2_ref[...]) * _BN, 0.0)
    z = jnp.dot(h, w3_ref[...], preferred_element_type=jnp.float32) + b3_ref[...]
    m = jnp.max(z, axis=1, keepdims=True)
    s = z - m
    out_ref[...] = s - jnp.log(jnp.sum(jnp.exp(s), axis=1, keepdims=True))


def _head_call(g, w1, b1, w2, b2, w3, b3):
    return pl.pallas_call(
        _head_body,
        out_shape=jax.ShapeDtypeStruct((B, w3.shape[1]), jnp.float32),
    )(g, w1, b1, w2, b2, w3, b3)


def _pad_k(w):
    return jnp.concatenate([w, jnp.zeros((8 - w.shape[0], w.shape[1]),
                                         w.dtype)], axis=0)


def kernel(pos, batch, params):
    p0 = pos.reshape(B, P, 3)
    x0 = p0[:, :, 0]
    y0 = p0[:, :, 1]
    z0 = p0[:, :, 2]

    cx1, cy1, cz1, cx2, cy2, cz2 = _fps2(x0, y0, z0, 512, 128)
    cent1 = jnp.stack([cx1, cy1, cz1], axis=-1)

    prow = jnp.transpose(p0, (0, 2, 1))
    pmat = jnp.concatenate([p0, jnp.zeros((B, P, 5), jnp.float32)], axis=-1)
    cmat1 = jnp.concatenate([cent1, jnp.zeros((B, 512, 5), jnp.float32)],
                            axis=-1)
    lt1 = (jnp.arange(P)[:, None] <= jnp.arange(P)[None, :]).astype(
        jnp.float32)

    (w1a, b1a), (w1b, b1b), (w1c, b1c) = params["sa1"]
    x1 = _sa_call(prow, pmat, cmat1, lt1,
                  _pad_k(w1a), _pad_k(w1a),
                  b1a[None], w1b, b1b[None], w1c, b1c[None],
                  nc_block=512, ns=32, r2=_R2_1)

    cent2 = jnp.stack([cx2, cy2, cz2], axis=-1)
    cmat2 = jnp.concatenate([cent2, jnp.zeros((B, 128, 5), jnp.float32)],
                            axis=-1)
    p1row = jnp.stack([cx1, cy1, cz1], axis=1)
    lt2 = (jnp.arange(512)[:, None] <= jnp.arange(512)[None, :]).astype(
        jnp.float32)

    (w2a, b2a), (w2b, b2b), (w2c, b2c) = params["sa2"]
    wx2 = w2a[:128]
    wr2 = _pad_k(w2a[128:])
    gfeat2 = jnp.concatenate([x1, cmat1], axis=-1)
    wg2 = jnp.concatenate([wx2, wr2], axis=0)
    x2 = _sa_call(p1row, gfeat2, cmat2, lt2,
                  wg2, wr2,
                  b2a[None], w2b, b2b[None], w2c, b2c[None],
                  nc_block=128, ns=64, r2=_R2_2)

    (w3a, b3a), (w3b, b3b), (w3c, b3c) = params["sa3"]
    wx3 = w3a[:256]
    wr3 = _pad_k(w3a[256:])
    g = _sa3_call(x2, cmat2, wx3, wr3, b3a[None], w3b, b3b[None],
                  w3c, b3c[None]).reshape(B, 1024)

    (wl1, bl1) = params["lin1"][0]
    (wl2, bl2) = params["lin2"][0]
    (wl3, bl3) = params["lin3"][0]
    return _head_call(g, wl1, bl1[None], wl2, bl2[None], wl3, bl3[None])


# speedup vs baseline: 1.3732x; 1.3732x over previous
import functools

import jax
import jax.numpy as jnp
import numpy as np
from jax import lax
from jax.experimental import pallas as pl
from jax.experimental.pallas import tpu as pltpu
from jax.experimental.pallas import tpu_sc as plsc

B = 32
P = 1024
_BN = np.float32(1.0 / np.sqrt(1.0 + 1e-5))
_R2_1 = np.float32(np.float64(0.2) * np.float64(0.2))
_R2_2 = np.float32(np.float64(0.4) * np.float64(0.4))


def _relu_bn(x):
    return jnp.maximum(x, 0.0) * _BN


_L = 16


def _fps_level(pn, npoint, xv, yv, zv, dv, cxv, cyv, czv):
    nchunk = pn // _L
    lane = lax.broadcasted_iota(jnp.int32, (_L,), 0)
    m0 = lane == 0
    zero = jnp.zeros((_L,), jnp.int32)
    for j in range(nchunk):
        dv[pl.ds(_L * j, _L)] = jnp.full((_L,), 1e10, jnp.float32)

    lx0 = plsc.load_gather(xv, [zero])
    ly0 = plsc.load_gather(yv, [zero])
    lz0 = plsc.load_gather(zv, [zero])
    plsc.store_scatter(cxv, [zero], lx0, mask=m0)
    plsc.store_scatter(cyv, [zero], ly0, mask=m0)
    plsc.store_scatter(czv, [zero], lz0, mask=m0)

    def outer(i, st):
        lxv, lyv, lzv = st
        vmax = jnp.full((_L,), -1.0, jnp.float32)
        vidx = jnp.zeros((_L,), jnp.int32)
        for j in range(nchunk):
            sl = pl.ds(_L * j, _L)
            dx = xv[sl] - lxv
            dy = yv[sl] - lyv
            dz = zv[sl] - lzv
            dd = (dx * dx + dy * dy) + dz * dz
            dn = jnp.minimum(dv[sl], dd)
            dv[sl] = dn
            better = dn > vmax
            vmax = jnp.where(better, dn, vmax)
            vidx = jnp.where(better, lane + _L * j, vidx)
        gm = jnp.max(vmax)
        biv = jnp.full((_L,), jnp.min(jnp.where(vmax == gm, vidx, pn)))
        nlx = plsc.load_gather(xv, [biv])
        nly = plsc.load_gather(yv, [biv])
        nlz = plsc.load_gather(zv, [biv])
        iv = jnp.full((_L,), i)
        plsc.store_scatter(cxv, [iv], nlx, mask=m0)
        plsc.store_scatter(cyv, [iv], nly, mask=m0)
        plsc.store_scatter(czv, [iv], nlz, mask=m0)
        return nlx, nly, nlz

    lax.fori_loop(1, npoint, outer, (lx0, ly0, lz0))


def _sc_fps_body(pn, n1, n2, x_hbm, y_hbm, z_hbm,
                 o1x_hbm, o1y_hbm, o1z_hbm, o2x_hbm, o2y_hbm, o2z_hbm,
                 xv, yv, zv, dv, c1x, c1y, c1z, d2v, c2x, c2y, c2z):
    wid = lax.axis_index("s") * 2 + lax.axis_index("c")
    pltpu.sync_copy(x_hbm.at[wid], xv)
    pltpu.sync_copy(y_hbm.at[wid], yv)
    pltpu.sync_copy(z_hbm.at[wid], zv)

    _fps_level(pn, n1, xv, yv, zv, dv, c1x, c1y, c1z)
    _fps_level(n1, n2, c1x, c1y, c1z, d2v, c2x, c2y, c2z)

    pltpu.sync_copy(c1x, o1x_hbm.at[wid])
    pltpu.sync_copy(c1y, o1y_hbm.at[wid])
    pltpu.sync_copy(c1z, o1z_hbm.at[wid])
    pltpu.sync_copy(c2x, o2x_hbm.at[wid])
    pltpu.sync_copy(c2y, o2y_hbm.at[wid])
    pltpu.sync_copy(c2z, o2z_hbm.at[wid])


def _fps2(x, y, z, n1, n2):
    pn = x.shape[1]
    out1 = jax.ShapeDtypeStruct((B, n1), jnp.float32)
    out2 = jax.ShapeDtypeStruct((B, n2), jnp.float32)
    mesh = plsc.VectorSubcoreMesh(core_axis_name="c", subcore_axis_name="s")
    fn = pl.kernel(
        functools.partial(_sc_fps_body, pn, n1, n2),
        out_type=(out1, out1, out1, out2, out2, out2),
        mesh=mesh,
        compiler_params=pltpu.CompilerParams(needs_layout_passes=False),
        scratch_types=[
            pltpu.VMEM((pn,), jnp.float32),
            pltpu.VMEM((pn,), jnp.float32),
            pltpu.VMEM((pn,), jnp.float32),
            pltpu.VMEM((pn,), jnp.float32),
            pltpu.VMEM((n1,), jnp.float32),
            pltpu.VMEM((n1,), jnp.float32),
            pltpu.VMEM((n1,), jnp.float32),
            pltpu.VMEM((n1,), jnp.float32),
            pltpu.VMEM((n2,), jnp.float32),
            pltpu.VMEM((n2,), jnp.float32),
            pltpu.VMEM((n2,), jnp.float32),
        ],
    )
    return fn(x, y, z)


def _sa_body(nc, pn, ns, r2, prow_ref, gfeat_ref, cmat_ref, lt_ref,
             wg_ref, wc_ref, b1_ref, w2_ref, b2_ref, w3_ref, b3_ref, out_ref):
    px = prow_ref[0, 0:1, :]
    py = prow_ref[0, 1:2, :]
    pz = prow_ref[0, 2:3, :]
    cm = cmat_ref[0]
    cx = cm[:, 0:1]
    cy = cm[:, 1:2]
    cz = cm[:, 2:3]
    d2 = ((cx - px) ** 2 + (cy - py) ** 2) + (cz - pz) ** 2
    mask = jnp.where(d2 <= r2, 1.0, 0.0)
    rank = jnp.dot(mask, lt_ref[...], preferred_element_type=jnp.float32)
    mrank = rank * mask
    count = rank[:, pn - 1 : pn]

    g_tab = jnp.dot(gfeat_ref[0], wg_ref[...],
                    preferred_element_type=jnp.float32)
    f1 = g_tab.shape[1]

    riota = lax.broadcasted_iota(jnp.int32, (1, ns, 1), 1).astype(
        jnp.float32) + 1.0
    sel = jnp.where(mrank[:, None, :] == riota, 1.0, 0.0)
    gath = jnp.dot(sel.reshape(nc * ns, pn), g_tab,
                   preferred_element_type=jnp.float32)
    g3 = gath.reshape(nc, ns, f1)
    slot = lax.broadcasted_iota(jnp.int32, (nc, ns, 1), 1).astype(jnp.float32)
    g3 = jnp.where(slot < count[:, :, None], g3, g3[:, 0:1, :])

    cc = jnp.dot(cm, wc_ref[...], preferred_element_type=jnp.float32)
    h = _relu_bn(g3 - cc[:, None, :] + b1_ref[...][None])
    h = h.reshape(nc * ns, f1)
    h = _relu_bn(jnp.dot(h, w2_ref[...],
                         preferred_element_type=jnp.float32) + b2_ref[...])
    h = _relu_bn(jnp.dot(h, w3_ref[...],
                         preferred_element_type=jnp.float32) + b3_ref[...])
    f3 = h.shape[1]
    out_ref[0] = jnp.max(h.reshape(nc, ns, f3), axis=1)


def _sa_call(prow, gfeat, cmat, lt, wg, wc, b1, w2, b2, w3, b3,
             nc_block, ns, r2):
    b, pn, cg = gfeat.shape
    ncent = cmat.shape[1]
    nblk = ncent // nc_block
    f3 = w3.shape[1]
    grid = (b, nblk)
    return pl.pallas_call(
        functools.partial(_sa_body, nc_block, pn, ns, r2),
        grid=grid,
        in_specs=[
            pl.BlockSpec((1, 3, pn), lambda i, j: (i, 0, 0)),
            pl.BlockSpec((1, pn, cg), lambda i, j: (i, 0, 0)),
            pl.BlockSpec((1, nc_block, 8), lambda i, j: (i, j, 0)),
            pl.BlockSpec((pn, pn), lambda i, j: (0, 0)),
            pl.BlockSpec(wg.shape, lambda i, j: (0, 0)),
            pl.BlockSpec(wc.shape, lambda i, j: (0, 0)),
            pl.BlockSpec(b1.shape, lambda i, j: (0, 0)),
            pl.BlockSpec(w2.shape, lambda i, j: (0, 0)),
            pl.BlockSpec(b2.shape, lambda i, j: (0, 0)),
            pl.BlockSpec(w3.shape, lambda i, j: (0, 0)),
            pl.BlockSpec(b3.shape, lambda i, j: (0, 0)),
        ],
        out_specs=pl.BlockSpec((1, nc_block, f3), lambda i, j: (i, j, 0)),
        out_shape=jax.ShapeDtypeStruct((b, ncent, f3), jnp.float32),
    )(prow, gfeat, cmat, lt, wg, wc, b1, w2, b2, w3, b3)


def _sa3_body(wx_ref, wr_ref, b1_ref, w2_ref, b2_ref, w3_ref, b3_ref,
              x2_ref, cmat_ref, out_ref):
    h = jnp.dot(x2_ref[0], wx_ref[...], preferred_element_type=jnp.float32)
    h = h + jnp.dot(cmat_ref[0], wr_ref[...],
                    preferred_element_type=jnp.float32)
    h = _relu_bn(h + b1_ref[...])
    h = _relu_bn(jnp.dot(h, w2_ref[...],
                         preferred_element_type=jnp.float32) + b2_ref[...])
    h = _relu_bn(jnp.dot(h, w3_ref[...],
                         preferred_element_type=jnp.float32) + b3_ref[...])
    out_ref[0] = jnp.max(h, axis=0, keepdims=True)


def _sa3_call(x2, cmat, wx, wr, b1, w2, b2, w3, b3):
    b, n2, _ = x2.shape
    return pl.pallas_call(
        _sa3_body,
        grid=(b,),
        in_specs=[
            pl.BlockSpec(wx.shape, lambda i: (0, 0)),
            pl.BlockSpec(wr.shape, lambda i: (0, 0)),
            pl.BlockSpec(b1.shape, lambda i: (0, 0)),
            pl.BlockSpec(w2.shape, lambda i: (0, 0)),
            pl.BlockSpec(b2.shape, lambda i: (0, 0)),
            pl.BlockSpec(w3.shape, lambda i: (0, 0)),
            pl.BlockSpec(b3.shape, lambda i: (0, 0)),
            pl.BlockSpec((1, n2, x2.shape[2]), lambda i: (i, 0, 0)),
            pl.BlockSpec((1, n2, 8), lambda i: (i, 0, 0)),
        ],
        out_specs=pl.BlockSpec((1, 1, 1024), lambda i: (i, 0, 0)),
        out_shape=jax.ShapeDtypeStruct((b, 1, 1024), jnp.float32),
    )(wx, wr, b1, w2, b2, w3, b3, x2, cmat)


def _head_body(g_ref, w1_ref, b1_ref, w2_ref, b2_ref, w3_ref, b3_ref,
               out_ref):
    h = jnp.maximum((jnp.dot(g_ref[...], w1_ref[...],
                             preferred_element_type=jnp.float32)
                     + b1_ref[...]) * _BN, 0.0)
    h = jnp.maximum((jnp.dot(h, w2_ref[...],
                             preferred_element_type=jnp.float32)
                     + b2_ref[...]) * _BN, 0.0)
    z = jnp.dot(h, w3_ref[...], preferred_element_type=jnp.float32) + b3_ref[...]
    m = jnp.max(z, axis=1, keepdims=True)
    s = z - m
    out_ref[...] = s - jnp.log(jnp.sum(jnp.exp(s), axis=1, keepdims=True))


def _head_call(g, w1, b1, w2, b2, w3, b3):
    return pl.pallas_call(
        _head_body,
        out_shape=jax.ShapeDtypeStruct((B, w3.shape[1]), jnp.float32),
    )(g, w1, b1, w2, b2, w3, b3)


def _pad_k(w):
    return jnp.concatenate([w, jnp.zeros((8 - w.shape[0], w.shape[1]),
                                         w.dtype)], axis=0)


def kernel(pos, batch, params):
    p0 = pos.reshape(B, P, 3)
    x0 = p0[:, :, 0]
    y0 = p0[:, :, 1]
    z0 = p0[:, :, 2]

    cx1, cy1, cz1, cx2, cy2, cz2 = _fps2(x0, y0, z0, 512, 128)
    cent1 = jnp.stack([cx1, cy1, cz1], axis=-1)

    prow = jnp.transpose(p0, (0, 2, 1))
    pmat = jnp.concatenate([p0, jnp.zeros((B, P, 5), jnp.float32)], axis=-1)
    cmat1 = jnp.concatenate([cent1, jnp.zeros((B, 512, 5), jnp.float32)],
                            axis=-1)
    lt1 = (jnp.arange(P)[:, None] <= jnp.arange(P)[None, :]).astype(
        jnp.float32)

    (w1a, b1a), (w1b, b1b), (w1c, b1c) = params["sa1"]
    x1 = _sa_call(prow, pmat, cmat1, lt1,
                  _pad_k(w1a), _pad_k(w1a),
                  b1a[None], w1b, b1b[None], w1c, b1c[None],
                  nc_block=512, ns=32, r2=_R2_1)

    cent2 = jnp.stack([cx2, cy2, cz2], axis=-1)
    cmat2 = jnp.concatenate([cent2, jnp.zeros((B, 128, 5), jnp.float32)],
                            axis=-1)
    p1row = jnp.stack([cx1, cy1, cz1], axis=1)
    lt2 = (jnp.arange(512)[:, None] <= jnp.arange(512)[None, :]).astype(
        jnp.float32)

    (w2a, b2a), (w2b, b2b), (w2c, b2c) = params["sa2"]
    wx2 = w2a[:128]
    wr2 = _pad_k(w2a[128:])
    gfeat2 = jnp.concatenate([x1, cmat1], axis=-1)
    wg2 = jnp.concatenate([wx2, wr2], axis=0)
    x2 = _sa_call(p1row, gfeat2, cmat2, lt2,
                  wg2, wr2,
                  b2a[None], w2b, b2b[None], w2c, b2c[None],
                  nc_block=128, ns=64, r2=_R2_2)

    (w3a, b3a), (w3b, b3b), (w3c, b3c) = params["sa3"]
    wx3 = w3a[:256]
    wr3 = _pad_k(w3a[256:])
    g = _sa3_call(x2, cmat2, wx3, wr3, b3a[None], w3b, b3b[None],
                  w3c, b3c[None]).reshape(B, 1024)

    (wl1, bl1) = params["lin1"][0]
    (wl2, bl2) = params["lin2"][0]
    (wl3, bl3) = params["lin3"][0]
    return _head_call(g, wl1, bl1[None], wl2, bl2[None], wl3, bl3[None])
